# baseline (speedup 1.0000x reference)
import functools

import jax
import jax.numpy as jnp
from jax import lax
from jax.experimental import pallas as pl
from jax.experimental.pallas import tpu as pltpu
from jax.experimental.pallas import tpu_sc as plsc

N_FACTORS = 32
N_ROWS = 1_000_000
BATCH = 16384
NC = 2
NS = 16
L = 16
NW = NC * NS
B_PER_W = BATCH // NW
N_CHUNK = B_PER_W // L

R_ELEMS = 4
N_ROUNDS = B_PER_W // R_ELEMS
HALF_ROUNDS = N_ROUNDS // 2
BLK_ROWS = R_ELEMS * N_FACTORS

R_TILES = -(-N_ROWS // 128)
BIAS_E = 128


def _sc_body(user_hbm, item_hbm, uft_hbm, itft_hbm, ub_hbm, ib_hbm, out_hbm,
             vidx_u, vidx_i, pb_u, pb_i, ublk, iblk,
             bb_u, bb_i, out_v, sem_a, sem_b, sem_s):
    wid = lax.axis_index("s") * NC + lax.axis_index("c")
    base = wid * B_PER_W
    lane = lax.iota(jnp.int32, L)

    pltpu.sync_copy(user_hbm.at[pl.ds(base, B_PER_W)], vidx_u)
    pltpu.sync_copy(item_hbm.at[pl.ds(base, B_PER_W)], vidx_i)

    def bidx(c, carry):
        sl = pl.ds(c * L, L)
        pb_u[sl] = vidx_u[sl] >> 7
        pb_i[sl] = vidx_i[sl] >> 7
        return carry

    lax.fori_loop(0, N_CHUNK, bidx, 0)

    def bias_pass(h, carry):
        e0 = h * BIAS_E
        cu = pltpu.make_async_copy(
            ub_hbm.at[pb_u.at[pl.ds(e0, BIAS_E)]], bb_u, sem_s)
        ci = pltpu.make_async_copy(
            ib_hbm.at[pb_i.at[pl.ds(e0, BIAS_E)]], bb_i, sem_s)
        cu.start()
        ci.start()
        cu.wait()
        ci.wait()

        def bx(c, carry2):
            sl = pl.ds(e0 + c * L, L)
            el16 = c * L + lane
            out_v[sl] = (plsc.load_gather(bb_u, [el16, vidx_u[sl] & 127]) +
                         plsc.load_gather(bb_i, [el16, vidx_i[sl] & 127]))
            return carry2

        lax.fori_loop(0, BIAS_E // L, bx, 0)
        return carry

    lax.fori_loop(0, B_PER_W // BIAS_E, bias_pass, 0)

    zeros16 = jnp.zeros((L,), jnp.int32)

    def fire(r, buf, sem):
        roff = buf * BLK_ROWS
        for el in range(R_ELEMS):
            e = r * R_ELEMS + el
            u = plsc.load_gather(vidx_u, [zeros16 + e])[0]
            i = plsc.load_gather(vidx_i, [zeros16 + e])[0]
            ua = pl.multiple_of((u >> 7) << 7, 128)
            ia = pl.multiple_of((i >> 7) << 7, 128)
            pltpu.make_async_copy(
                uft_hbm.at[:, pl.ds(ua, 128)],
                ublk.at[pl.ds(roff + el * N_FACTORS, N_FACTORS)], sem).start()
            pltpu.make_async_copy(
                itft_hbm.at[:, pl.ds(ia, 128)],
                iblk.at[pl.ds(roff + el * N_FACTORS, N_FACTORS)], sem).start()

    def drain(buf, sem):
        roff = buf * BLK_ROWS
        pltpu.make_async_copy(
            uft_hbm.at[:, pl.ds(0, 128 * R_ELEMS)],
            ublk.at[pl.ds(roff, BLK_ROWS)], sem).wait()
        pltpu.make_async_copy(
            uft_hbm.at[:, pl.ds(0, 128 * R_ELEMS)],
            iblk.at[pl.ds(roff, BLK_ROWS)], sem).wait()

    def extract(r, buf):
        roff = buf * BLK_ROWS
        c = r // (L // R_ELEMS)
        lbase = (r % (L // R_ELEMS)) * R_ELEMS
        contrib = jnp.zeros((L,), jnp.float32)
        for el in range(R_ELEMS):
            e = r * R_ELEMS + el
            lu = plsc.load_gather(vidx_u, [zeros16 + e]) & 127
            li = plsc.load_gather(vidx_i, [zeros16 + e]) & 127
            rows = roff + el * N_FACTORS + lane
            p = (plsc.load_gather(ublk, [rows, lu]) *
                 plsc.load_gather(iblk, [rows, li]))
            p = p + (plsc.load_gather(ublk, [rows + L, lu]) *
                     plsc.load_gather(iblk, [rows + L, li]))
            s = lax.reduce_sum_p.bind(p, axes=(0,))
            contrib = jnp.where(lane == lbase + el, s, contrib)
        sl = pl.ds(c * L, L)
        out_v[sl] = out_v[sl] + contrib

    fire(0, 0, sem_a)

    def steady(rp, carry):
        ra = 2 * rp
        rb = 2 * rp + 1
        fire(rb, 1, sem_b)
        drain(0, sem_a)
        extract(ra, 0)

        @pl.when(rp < HALF_ROUNDS - 1)
        def _():
            fire(ra + 2, 0, sem_a)

        drain(1, sem_b)
        extract(rb, 1)
        return carry

    lax.fori_loop(0, HALF_ROUNDS, steady, 0)

    pltpu.sync_copy(out_v, out_hbm.at[pl.ds(base, B_PER_W)])


@jax.jit
def _predict(user, item, user_factors, item_factors, user_biases, item_biases):
    run = pl.kernel(
        _sc_body,
        out_type=jax.ShapeDtypeStruct((BATCH,), jnp.float32),
        mesh=plsc.VectorSubcoreMesh(core_axis_name="c", subcore_axis_name="s"),
        compiler_params=pltpu.CompilerParams(needs_layout_passes=False),
        scratch_types=[
            pltpu.VMEM((B_PER_W,), jnp.int32),
            pltpu.VMEM((B_PER_W,), jnp.int32),
            pltpu.VMEM((B_PER_W,), jnp.int32),
            pltpu.VMEM((B_PER_W,), jnp.int32),
            pltpu.VMEM((2 * BLK_ROWS, 128), jnp.float32),
            pltpu.VMEM((2 * BLK_ROWS, 128), jnp.float32),
            pltpu.VMEM((BIAS_E, 128), jnp.float32),
            pltpu.VMEM((BIAS_E, 128), jnp.float32),
            pltpu.VMEM((B_PER_W,), jnp.float32),
            pltpu.SemaphoreType.DMA,
            pltpu.SemaphoreType.DMA,
            pltpu.SemaphoreType.DMA,
        ],
    )
    pad = R_TILES * 128 - N_ROWS
    ub2 = jnp.pad(user_biases.reshape(-1), (0, pad)).reshape(R_TILES, 128)
    ib2 = jnp.pad(item_biases.reshape(-1), (0, pad)).reshape(R_TILES, 128)
    return run(user, item, user_factors.T, item_factors.T, ub2, ib2)


def kernel(user, item, user_factors, item_factors, user_biases, item_biases):
    return _predict(user, item, user_factors, item_factors,
                    user_biases, item_biases)


# speedup vs baseline: 1.0412x; 1.0412x over previous
import functools

import jax
import jax.numpy as jnp
from jax import lax
from jax.experimental import pallas as pl
from jax.experimental.pallas import tpu as pltpu
from jax.experimental.pallas import tpu_sc as plsc

N_FACTORS = 32
N_ROWS = 1_000_000
BATCH = 16384
NC = 2
NS = 16
L = 16
NW = NC * NS
B_PER_W = BATCH // NW
N_CHUNK = B_PER_W // L

R_ELEMS = 4
N_ROUNDS = B_PER_W // R_ELEMS
HALF_ROUNDS = N_ROUNDS // 2
BLK_ROWS = R_ELEMS * N_FACTORS


def _sc_body(user_hbm, item_hbm, uft_hbm, itft_hbm, ub_hbm, ib_hbm, out_hbm,
             vidx_u, vidx_i, ublk, iblk, bblk_u, bblk_i, out_v,
             sem_a, sem_b, sem_s):
    wid = lax.axis_index("s") * NC + lax.axis_index("c")
    base = wid * B_PER_W
    lane = lax.iota(jnp.int32, L)
    zeros16 = jnp.zeros((L,), jnp.int32)
    zf16 = jnp.zeros((L,), jnp.float32)

    pltpu.sync_copy(user_hbm.at[pl.ds(base, B_PER_W)], vidx_u)
    pltpu.sync_copy(item_hbm.at[pl.ds(base, B_PER_W)], vidx_i)

    def zinit(c, carry):
        out_v[pl.ds(c * L, L)] = zf16
        return carry

    lax.fori_loop(0, N_CHUNK, zinit, 0)

    def fire(r, buf, sem):
        roff = buf * BLK_ROWS
        for el in range(R_ELEMS):
            e = r * R_ELEMS + el
            u = plsc.load_gather(vidx_u, [zeros16 + e])[0]
            i = plsc.load_gather(vidx_i, [zeros16 + e])[0]
            ua = pl.multiple_of((u >> 7) << 7, 128)
            ia = pl.multiple_of((i >> 7) << 7, 128)
            pltpu.make_async_copy(
                uft_hbm.at[:, pl.ds(ua, 128)],
                ublk.at[pl.ds(roff + el * N_FACTORS, N_FACTORS)], sem).start()
            pltpu.make_async_copy(
                itft_hbm.at[:, pl.ds(ia, 128)],
                iblk.at[pl.ds(roff + el * N_FACTORS, N_FACTORS)], sem).start()
            ub8 = pl.multiple_of((u >> 3) << 3, 8)
            ib8 = pl.multiple_of((i >> 3) << 3, 8)
            pltpu.make_async_copy(
                ub_hbm.at[pl.ds(ub8, 8)], bblk_u.at[pl.ds(e * 8, 8)],
                sem_s).start()
            pltpu.make_async_copy(
                ib_hbm.at[pl.ds(ib8, 8)], bblk_i.at[pl.ds(e * 8, 8)],
                sem_s).start()

    def drain(buf, sem):
        roff = buf * BLK_ROWS
        pltpu.make_async_copy(
            uft_hbm.at[:, pl.ds(0, 128 * R_ELEMS)],
            ublk.at[pl.ds(roff, BLK_ROWS)], sem).wait()
        pltpu.make_async_copy(
            uft_hbm.at[:, pl.ds(0, 128 * R_ELEMS)],
            iblk.at[pl.ds(roff, BLK_ROWS)], sem).wait()

    def extract(r, buf):
        roff = buf * BLK_ROWS
        c = r // (L // R_ELEMS)
        lbase = (r % (L // R_ELEMS)) * R_ELEMS
        contrib = zf16
        for el in range(R_ELEMS):
            e = r * R_ELEMS + el
            lu = plsc.load_gather(vidx_u, [zeros16 + e]) & 127
            li = plsc.load_gather(vidx_i, [zeros16 + e]) & 127
            rows = roff + el * N_FACTORS + lane
            p = (plsc.load_gather(ublk, [rows, lu]) *
                 plsc.load_gather(iblk, [rows, li]))
            p = p + (plsc.load_gather(ublk, [rows + L, lu]) *
                     plsc.load_gather(iblk, [rows + L, li]))
            s = lax.reduce_sum_p.bind(p, axes=(0,))
            contrib = jnp.where(lane == lbase + el, s, contrib)
        sl = pl.ds(c * L, L)
        out_v[sl] = out_v[sl] + contrib

    fire(0, 0, sem_a)

    def steady(rp, carry):
        ra = 2 * rp
        rb = 2 * rp + 1
        fire(rb, 1, sem_b)
        drain(0, sem_a)
        extract(ra, 0)

        @pl.when(rp < HALF_ROUNDS - 1)
        def _():
            fire(ra + 2, 0, sem_a)

        drain(1, sem_b)
        extract(rb, 1)
        return carry

    lax.fori_loop(0, HALF_ROUNDS, steady, 0)

    pltpu.make_async_copy(ub_hbm.at[pl.ds(0, B_PER_W * 8)], bblk_u,
                          sem_s).wait()
    pltpu.make_async_copy(ub_hbm.at[pl.ds(0, B_PER_W * 8)], bblk_i,
                          sem_s).wait()

    def badd(c, carry):
        sl = pl.ds(c * L, L)
        eb = (c * L + lane) * 8
        bu = plsc.load_gather(bblk_u, [eb + (vidx_u[sl] & 7)])
        bi = plsc.load_gather(bblk_i, [eb + (vidx_i[sl] & 7)])
        out_v[sl] = out_v[sl] + bu + bi
        return carry

    lax.fori_loop(0, N_CHUNK, badd, 0)

    pltpu.sync_copy(out_v, out_hbm.at[pl.ds(base, B_PER_W)])


@jax.jit
def _predict(user, item, user_factors, item_factors, user_biases, item_biases):
    run = pl.kernel(
        _sc_body,
        out_type=jax.ShapeDtypeStruct((BATCH,), jnp.float32),
        mesh=plsc.VectorSubcoreMesh(core_axis_name="c", subcore_axis_name="s"),
        compiler_params=pltpu.CompilerParams(needs_layout_passes=False),
        scratch_types=[
            pltpu.VMEM((B_PER_W,), jnp.int32),
            pltpu.VMEM((B_PER_W,), jnp.int32),
            pltpu.VMEM((2 * BLK_ROWS, 128), jnp.float32),
            pltpu.VMEM((2 * BLK_ROWS, 128), jnp.float32),
            pltpu.VMEM((B_PER_W * 8,), jnp.float32),
            pltpu.VMEM((B_PER_W * 8,), jnp.float32),
            pltpu.VMEM((B_PER_W,), jnp.float32),
            pltpu.SemaphoreType.DMA,
            pltpu.SemaphoreType.DMA,
            pltpu.SemaphoreType.DMA,
        ],
    )
    return run(user, item, user_factors.T, item_factors.T,
               user_biases.reshape(-1), item_biases.reshape(-1))


def kernel(user, item, user_factors, item_factors, user_biases, item_biases):
    return _predict(user, item, user_factors, item_factors,
                    user_biases, item_biases)
